# Initial kernel scaffold; baseline (speedup 1.0000x reference)
import jax
import jax.numpy as jnp
from jax.experimental import pallas as pl


def kernel(indices, tables):
    raise NotImplementedError("write your pallas kernel here")


# speedup vs baseline: 8.6654x; 8.6654x over previous
import jax
import jax.numpy as jnp
from jax import lax
from jax.experimental import pallas as pl
from jax.experimental.pallas import tpu as pltpu
from jax.experimental.pallas import tpu_sc as plsc

N_FIELDS = 26
VOCAB = 100000
EMB_DIM = 32
BATCH = 16384

NC = 2
NS = 16
NW = NC * NS
TOTAL = BATCH * N_FIELDS
PER_W = TOTAL // NW
CHUNK = 128
NCH = PER_W // CHUNK


def _gather_body(idx_hbm, tab_hbm, out_hbm, idx_v, rows_v, sem):
    wid = lax.axis_index("s") * NC + lax.axis_index("c")

    def chunk(c, carry):
        cid = wid * NCH + c
        pltpu.sync_copy(idx_hbm.at[cid], idx_v)
        pltpu.async_copy(tab_hbm.at[idx_v], rows_v, sem).wait()
        pltpu.sync_copy(rows_v, out_hbm.at[pl.ds(cid * CHUNK, CHUNK)])
        return carry

    lax.fori_loop(0, NCH, chunk, 0)


def kernel(indices, tables):
    flat_tables = tables.reshape(N_FIELDS * VOCAB, EMB_DIM)
    offsets = jnp.arange(N_FIELDS, dtype=jnp.int32) * VOCAB
    flat_idx = (indices.astype(jnp.int32) + offsets[None, :]).reshape(
        TOTAL // CHUNK, CHUNK)

    mesh = plsc.VectorSubcoreMesh(core_axis_name="c", subcore_axis_name="s")
    out = pl.kernel(
        _gather_body,
        mesh=mesh,
        out_type=jax.ShapeDtypeStruct((TOTAL, EMB_DIM), jnp.float32),
        scratch_types=[
            pltpu.VMEM((CHUNK,), jnp.int32),
            pltpu.VMEM((CHUNK, EMB_DIM), jnp.float32),
            pltpu.SemaphoreType.DMA,
        ],
        compiler_params=pltpu.CompilerParams(use_tc_tiling_on_sc=False),
    )(flat_idx, flat_tables)
    return out.reshape(BATCH, N_FIELDS * EMB_DIM)


# speedup vs baseline: 9.4573x; 1.0914x over previous
import jax
import jax.numpy as jnp
from jax import lax
from jax.experimental import pallas as pl
from jax.experimental.pallas import tpu as pltpu
from jax.experimental.pallas import tpu_sc as plsc

N_FIELDS = 26
VOCAB = 100000
EMB_DIM = 32
BATCH = 16384

NC = 2
NS = 16
NW = NC * NS
TOTAL = BATCH * N_FIELDS
PER_W = TOTAL // NW
CHUNK = 128
NCH = PER_W // CHUNK
K = 4
NGRP = NCH // K
NPAIR = NGRP // 2


def _gather_body(idx_hbm, tab_hbm, out_hbm, idx_v, rows_v,
                 gsem0, gsem1, wsem0, wsem1):
    wid = lax.axis_index("s") * NC + lax.axis_index("c")
    base_chunk = wid * NCH

    def fire_gathers(grp, set_, sem):
        for b in range(K):
            pltpu.async_copy(tab_hbm.at[idx_v.at[grp * K + b]],
                             rows_v.at[set_, b], sem)

    def wait_gathers(set_, sem):
        for b in range(K):
            pltpu.make_async_copy(tab_hbm.at[idx_v.at[0]],
                                  rows_v.at[set_, b], sem).wait()

    def fire_wouts(grp, set_, sem):
        for b in range(K):
            c = base_chunk + grp * K + b
            pltpu.async_copy(rows_v.at[set_, b],
                             out_hbm.at[pl.ds(c * CHUNK, CHUNK)], sem)

    def wait_wouts(set_, sem):
        for b in range(K):
            pltpu.make_async_copy(rows_v.at[set_, b],
                                  out_hbm.at[pl.ds(0, CHUNK)], sem).wait()

    pltpu.sync_copy(idx_hbm.at[pl.ds(wid * NCH, NCH)], idx_v)
    fire_gathers(0, 0, gsem0)

    def pair(j, carry):
        g0 = 2 * j
        g1 = 2 * j + 1

        @pl.when(j > 0)
        def _():
            wait_wouts(1, wsem1)
        fire_gathers(g1, 1, gsem1)
        wait_gathers(0, gsem0)
        fire_wouts(g0, 0, wsem0)
        wait_wouts(0, wsem0)

        @pl.when(g0 + 2 < NGRP)
        def _():
            fire_gathers(g0 + 2, 0, gsem0)
        wait_gathers(1, gsem1)
        fire_wouts(g1, 1, wsem1)
        return carry

    lax.fori_loop(0, NPAIR, pair, 0)
    wait_wouts(1, wsem1)


def kernel(indices, tables):
    flat_tables = tables.reshape(N_FIELDS * VOCAB, EMB_DIM)
    offsets = jnp.arange(N_FIELDS, dtype=jnp.int32) * VOCAB
    flat_idx = (indices.astype(jnp.int32) + offsets[None, :]).reshape(
        TOTAL // CHUNK, CHUNK)

    mesh = plsc.VectorSubcoreMesh(core_axis_name="c", subcore_axis_name="s")
    out = pl.kernel(
        _gather_body,
        mesh=mesh,
        out_type=jax.ShapeDtypeStruct((TOTAL, EMB_DIM), jnp.float32),
        scratch_types=[
            pltpu.VMEM((NCH, CHUNK), jnp.int32),
            pltpu.VMEM((2, K, CHUNK, EMB_DIM), jnp.float32),
            pltpu.SemaphoreType.DMA,
            pltpu.SemaphoreType.DMA,
            pltpu.SemaphoreType.DMA,
            pltpu.SemaphoreType.DMA,
        ],
        compiler_params=pltpu.CompilerParams(use_tc_tiling_on_sc=False),
    )(flat_idx, flat_tables)
    return out.reshape(BATCH, N_FIELDS * EMB_DIM)


# speedup vs baseline: 9.4698x; 1.0013x over previous
import jax
import jax.numpy as jnp
from jax import lax
from jax.experimental import pallas as pl
from jax.experimental.pallas import tpu as pltpu
from jax.experimental.pallas import tpu_sc as plsc

N_FIELDS = 26
VOCAB = 100000
EMB_DIM = 32
BATCH = 16384

NC = 2
NS = 16
NW = NC * NS
TOTAL = BATCH * N_FIELDS
PER_W = TOTAL // NW
CHUNK = 256
NCH = PER_W // CHUNK
K = 2
NGRP = NCH // K
NPAIR = NGRP // 2


def _gather_body(idx_hbm, tab_hbm, out_hbm, idx_v, rows_v,
                 gsem0, gsem1, wsem0, wsem1):
    wid = lax.axis_index("s") * NC + lax.axis_index("c")
    base_chunk = wid * NCH

    def fire_gathers(grp, set_, sem):
        for b in range(K):
            pltpu.async_copy(tab_hbm.at[idx_v.at[grp * K + b]],
                             rows_v.at[set_, b], sem)

    def wait_gathers(set_, sem):
        for b in range(K):
            pltpu.make_async_copy(tab_hbm.at[idx_v.at[0]],
                                  rows_v.at[set_, b], sem).wait()

    def fire_wouts(grp, set_, sem):
        for b in range(K):
            c = base_chunk + grp * K + b
            pltpu.async_copy(rows_v.at[set_, b],
                             out_hbm.at[pl.ds(c * CHUNK, CHUNK)], sem)

    def wait_wouts(set_, sem):
        for b in range(K):
            pltpu.make_async_copy(rows_v.at[set_, b],
                                  out_hbm.at[pl.ds(0, CHUNK)], sem).wait()

    pltpu.sync_copy(idx_hbm.at[pl.ds(wid * NCH, NCH)], idx_v)
    fire_gathers(0, 0, gsem0)

    def pair(j, carry):
        g0 = 2 * j
        g1 = 2 * j + 1

        @pl.when(j > 0)
        def _():
            wait_wouts(1, wsem1)
        fire_gathers(g1, 1, gsem1)
        wait_gathers(0, gsem0)
        fire_wouts(g0, 0, wsem0)
        wait_wouts(0, wsem0)

        @pl.when(g0 + 2 < NGRP)
        def _():
            fire_gathers(g0 + 2, 0, gsem0)
        wait_gathers(1, gsem1)
        fire_wouts(g1, 1, wsem1)
        return carry

    lax.fori_loop(0, NPAIR, pair, 0)
    wait_wouts(1, wsem1)


def kernel(indices, tables):
    flat_tables = tables.reshape(N_FIELDS * VOCAB, EMB_DIM)
    offsets = jnp.arange(N_FIELDS, dtype=jnp.int32) * VOCAB
    flat_idx = (indices.astype(jnp.int32) + offsets[None, :]).reshape(
        TOTAL // CHUNK, CHUNK)

    mesh = plsc.VectorSubcoreMesh(core_axis_name="c", subcore_axis_name="s")
    out = pl.kernel(
        _gather_body,
        mesh=mesh,
        out_type=jax.ShapeDtypeStruct((TOTAL, EMB_DIM), jnp.float32),
        scratch_types=[
            pltpu.VMEM((NCH, CHUNK), jnp.int32),
            pltpu.VMEM((2, K, CHUNK, EMB_DIM), jnp.float32),
            pltpu.SemaphoreType.DMA,
            pltpu.SemaphoreType.DMA,
            pltpu.SemaphoreType.DMA,
            pltpu.SemaphoreType.DMA,
        ],
        compiler_params=pltpu.CompilerParams(use_tc_tiling_on_sc=False),
    )(flat_idx, flat_tables)
    return out.reshape(BATCH, N_FIELDS * EMB_DIM)
